# baseline (speedup 1.0000x reference)
import jax
import jax.numpy as jnp
from jax import lax
from jax.experimental import pallas as pl
from jax.experimental.pallas import tpu as pltpu

_K = 128
_NB = 8
_F = 128
_C = 96
_H = 224
_W = 224
_N = _H * _W
_D = 128
_TILE = 512
_NTILES = _N // _TILE
_ITERS = 10

_PREC = None


def _kmeans_graph_kernel(x_ref, c0_ref, w_ref, b_ref, feats_ref, adj_ref,
                         c_ref, oh_ref):
    f32 = jnp.float32
    lane = lax.broadcasted_iota(jnp.int32, (_K, _D), 1)
    dim_mask = (lane < 99).astype(f32)
    coord_mask = ((lane >= 96) & (lane < 99)).astype(f32)
    lane_t = lax.broadcasted_iota(jnp.int32, (_TILE, _D), 1)
    dmask_t = (lane_t < 99).astype(f32)

    c_ref[...] = c0_ref[...] * dim_mask

    def km_iter(_, carry):
        c = c_ref[...]
        b2 = jnp.sum(c * c, axis=1)

        def tile_body(t, carry2):
            xt = x_ref[pl.ds(t * _TILE, _TILE), :]
            xm = xt * dmask_t
            a2 = jnp.sum(xm * xm, axis=1, keepdims=True)
            scores = lax.dot_general(
                xt, c, (((1,), (1,)), ((), ())), precision=_PREC)
            d2 = a2 + b2[None, :] - 2.0 * scores
            dists = jnp.sqrt(jnp.maximum(d2, 0.0))
            m = jnp.min(dists, axis=1, keepdims=True)
            ii = lax.broadcasted_iota(jnp.int32, (_TILE, _K), 1)
            labels = jnp.min(jnp.where(dists == m, ii, _K), axis=1)
            oh_ref[pl.ds(t * _TILE, _TILE), :] = (
                labels[:, None] == ii).astype(f32)
            return carry2

        lax.fori_loop(0, _NTILES, tile_body, 0)

        sums = lax.dot_general(
            oh_ref[...], x_ref[...], (((0,), (0,)), ((), ())),
            precision=_PREC)
        counts = jnp.clip(sums[:, 99:100], 1.0, None)
        c_ref[...] = (sums / counts) * dim_mask
        return carry

    lax.fori_loop(0, _ITERS, km_iter, 0)

    c = c_ref[...]
    feats_ref[...] = lax.dot_general(
        c, w_ref[...], (((1,), (1,)), ((), ())), precision=_PREC) + b_ref[...]

    cc = c * coord_mask
    ccn = jnp.sum(cc * cc, axis=1)
    g = lax.dot_general(cc, cc, (((1,), (1,)), ((), ())), precision=_PREC)
    d2 = ccn[:, None] + ccn[None, :] - 2.0 * g
    dist = jnp.sqrt(jnp.maximum(d2, 0.0))
    kk = lax.broadcasted_iota(jnp.int32, (_K, _K), 1)

    def knn_body(_, carry2):
        dd, adj = carry2
        m = jnp.min(dd, axis=1, keepdims=True)
        am = jnp.min(jnp.where(dd == m, kk, _K), axis=1)
        sel = am[:, None] == kk
        adj = jnp.where(sel, 1.0, adj)
        dd = jnp.where(sel, jnp.inf, dd)
        return dd, adj

    _, adj = lax.fori_loop(0, _NB, knn_body,
                           (dist, jnp.zeros((_K, _K), f32)))
    adj_ref[...] = ((adj + adj.T) > 0.0).astype(f32)


def kernel(feature_map, W, b):
    f32 = jnp.float32
    fm = feature_map[:, :, 0, :, :]
    B = fm.shape[0]

    y = jnp.linspace(0.0, 1.0, _H)
    x = jnp.linspace(0.0, 1.0, _W)
    gy, gx = jnp.meshgrid(y, x, indexing='ij')
    gz = jnp.full((_H, _W), 0.5, dtype=f32)
    coords = jnp.stack([gz, gy, gx], axis=0).reshape(3, _N) * 5.0

    Wp = jnp.zeros((_F, _D), f32).at[:, :99].set(W)
    bp = b.reshape(1, _F).astype(f32)

    feats_list = []
    adj_list = []
    for i in range(B):
        fm_flat = fm[i].reshape(_C, _N)
        xt = jnp.concatenate([fm_flat, coords], axis=0).T
        xp = jnp.zeros((_N, _D), f32).at[:, :99].set(xt).at[:, 99].set(1.0)
        idx = jax.random.permutation(jax.random.key(42 + i), _N)[:_K]
        c0 = xp[idx]

        feats, adj = pl.pallas_call(
            _kmeans_graph_kernel,
            out_shape=(
                jax.ShapeDtypeStruct((_K, _F), f32),
                jax.ShapeDtypeStruct((_K, _K), f32),
            ),
            scratch_shapes=[
                pltpu.VMEM((_K, _D), f32),
                pltpu.VMEM((_N, _D), f32),
            ],
        )(xp, c0, Wp, bp)
        feats_list.append(feats)
        adj_list.append(adj)
    return jnp.stack(feats_list), jnp.stack(adj_list)


# speedup vs baseline: 1.3369x; 1.3369x over previous
import jax
import jax.numpy as jnp
from jax import lax
from jax.experimental import pallas as pl
from jax.experimental.pallas import tpu as pltpu

_K = 128
_NB = 8
_F = 128
_C = 96
_H = 224
_W = 224
_N = _H * _W
_D = 128
_TILE = 7168
_NTILES = _N // _TILE
_ITERS = 10

_PREC = None


def _kmeans_graph_kernel(x_ref, c0_ref, w_ref, b_ref, feats_ref, adj_ref,
                         c_ref, oh_ref):
    f32 = jnp.float32
    lane = lax.broadcasted_iota(jnp.int32, (_K, _D), 1)
    dim_mask = (lane < 99).astype(f32)
    coord_mask = ((lane >= 96) & (lane < 99)).astype(f32)
    lane_t = lax.broadcasted_iota(jnp.int32, (_TILE, _D), 1)
    dmask_t = (lane_t < 99).astype(f32)

    c_ref[...] = c0_ref[...] * dim_mask

    def km_iter(_, carry):
        c = c_ref[...]
        b2 = jnp.sum(c * c, axis=1)

        def tile_body(t, carry2):
            xt = x_ref[pl.ds(t * _TILE, _TILE), :]
            xm = xt * dmask_t
            a2 = jnp.sum(xm * xm, axis=1, keepdims=True)
            scores = lax.dot_general(
                xt, c, (((1,), (1,)), ((), ())), precision=_PREC)
            d2 = a2 + b2[None, :] - 2.0 * scores
            dists = jnp.sqrt(jnp.maximum(d2, 0.0))
            m = jnp.min(dists, axis=1, keepdims=True)
            ii = lax.broadcasted_iota(jnp.int32, (_TILE, _K), 1)
            labels = jnp.min(jnp.where(dists == m, ii, _K), axis=1)
            oh_ref[pl.ds(t * _TILE, _TILE), :] = (
                labels[:, None] == ii).astype(f32)
            return carry2

        lax.fori_loop(0, _NTILES, tile_body, 0)

        sums = lax.dot_general(
            oh_ref[...], x_ref[...], (((0,), (0,)), ((), ())),
            precision=_PREC)
        counts = jnp.clip(sums[:, 99:100], 1.0, None)
        c_ref[...] = (sums / counts) * dim_mask
        return carry

    lax.fori_loop(0, _ITERS, km_iter, 0)

    c = c_ref[...]
    feats_ref[...] = lax.dot_general(
        c, w_ref[...], (((1,), (1,)), ((), ())), precision=_PREC) + b_ref[...]

    cc = c * coord_mask
    ccn = jnp.sum(cc * cc, axis=1)
    g = lax.dot_general(cc, cc, (((1,), (1,)), ((), ())), precision=_PREC)
    d2 = ccn[:, None] + ccn[None, :] - 2.0 * g
    dist = jnp.sqrt(jnp.maximum(d2, 0.0))
    kk = lax.broadcasted_iota(jnp.int32, (_K, _K), 1)

    def knn_body(_, carry2):
        dd, adj = carry2
        m = jnp.min(dd, axis=1, keepdims=True)
        am = jnp.min(jnp.where(dd == m, kk, _K), axis=1)
        sel = am[:, None] == kk
        adj = jnp.where(sel, 1.0, adj)
        dd = jnp.where(sel, jnp.inf, dd)
        return dd, adj

    _, adj = lax.fori_loop(0, _NB, knn_body,
                           (dist, jnp.zeros((_K, _K), f32)))
    adj_ref[...] = ((adj + adj.T) > 0.0).astype(f32)


def kernel(feature_map, W, b):
    f32 = jnp.float32
    fm = feature_map[:, :, 0, :, :]
    B = fm.shape[0]

    y = jnp.linspace(0.0, 1.0, _H)
    x = jnp.linspace(0.0, 1.0, _W)
    gy, gx = jnp.meshgrid(y, x, indexing='ij')
    gz = jnp.full((_H, _W), 0.5, dtype=f32)
    coords = jnp.stack([gz, gy, gx], axis=0).reshape(3, _N) * 5.0

    Wp = jnp.zeros((_F, _D), f32).at[:, :99].set(W)
    bp = b.reshape(1, _F).astype(f32)

    feats_list = []
    adj_list = []
    for i in range(B):
        fm_flat = fm[i].reshape(_C, _N)
        xt = jnp.concatenate([fm_flat, coords], axis=0).T
        xp = jnp.zeros((_N, _D), f32).at[:, :99].set(xt).at[:, 99].set(1.0)
        idx = jax.random.permutation(jax.random.key(42 + i), _N)[:_K]
        c0 = xp[idx]

        feats, adj = pl.pallas_call(
            _kmeans_graph_kernel,
            out_shape=(
                jax.ShapeDtypeStruct((_K, _F), f32),
                jax.ShapeDtypeStruct((_K, _K), f32),
            ),
            scratch_shapes=[
                pltpu.VMEM((_K, _D), f32),
                pltpu.VMEM((_N, _D), f32),
            ],
        )(xp, c0, Wp, bp)
        feats_list.append(feats)
        adj_list.append(adj)
    return jnp.stack(feats_list), jnp.stack(adj_list)
